# baseline (device time: 222564 ns/iter reference)
import jax
import jax.numpy as jnp
from jax import lax
from jax.experimental import pallas as pl
from jax.experimental.pallas import tpu as pltpu

N_DEV = 4
M_PER = 2048
K = 8192
N_PER = 1024
K_BLK = 1024
N_K = K // K_BLK
X_BLK = 256
N_KX = K // X_BLK


def kernel(x, w_mat):
    def body(x_hbm, w_hbm, out_ref, x_vmem, xf32_bufs, w_bufs, acc_ref,
             send_bufs, x_sems, w_sems, send_sems, recv_sems, local_sem):
        my = lax.axis_index("i")

        barrier = pltpu.get_barrier_semaphore()
        for d in range(1, N_DEV):
            pl.semaphore_signal(
                barrier, inc=1,
                device_id=((my + d) % N_DEV,),
                device_id_type=pl.DeviceIdType.MESH,
            )
        pl.semaphore_wait(barrier, N_DEV - 1)

        def x_load(ks, b):
            return pltpu.make_async_copy(
                x_hbm.at[:, pl.ds(ks * X_BLK, X_BLK)],
                xf32_bufs.at[b],
                x_sems.at[b],
            )

        def w_load(col, k, b):
            return pltpu.make_async_copy(
                w_hbm.at[pl.ds(k * K_BLK, K_BLK), pl.ds(col * N_PER, N_PER)],
                w_bufs.at[b],
                w_sems.at[b],
            )

        def my_out_rows():
            return out_ref.at[pl.ds(my * M_PER, M_PER), :]

        cols = [(my + d) % N_DEV for d in range(1, N_DEV)] + [my]
        rdma_by_slot = {}
        for idx in range(N_DEV):
            col = cols[idx]
            if idx == 0:
                w_load(col, 0, 0).start()
                w_load(col, 1, 1).start()
                x_load(0, 0).start()
                x_load(1, 1).start()
            for mi in range(2):
                rows = slice(mi * 1024, (mi + 1) * 1024)
                acc_ref[rows, :] = jnp.zeros((1024, N_PER), jnp.float32)

            def jbody(j, _, idx=idx, col=col):
                for b in (0, 1):
                    k = 2 * j + b
                    if idx == 0:
                        for sb in range(K_BLK // X_BLK):
                            ks = k * (K_BLK // X_BLK) + sb
                            xb = sb % 2
                            x_load(ks, xb).wait()
                            x_vmem[:, pl.ds(ks * X_BLK, X_BLK)] = (
                                xf32_bufs[xb].astype(jnp.bfloat16)
                            )

                            @pl.when(ks + 2 < N_KX)
                            def _():
                                x_load(ks + 2, xb).start()

                    w_load(col, k, b).wait()
                    wt = w_bufs[b].astype(jnp.bfloat16)
                    for mi in range(2):
                        rows = slice(mi * 1024, (mi + 1) * 1024)
                        acc_ref[rows, :] += lax.dot_general(
                            x_vmem[rows, pl.ds(k * K_BLK, K_BLK)],
                            wt,
                            (((1,), (0,)), ((), ())),
                            preferred_element_type=jnp.float32,
                        )

                    @pl.when(k + 2 < N_K)
                    def _():
                        w_load(col, k + 2, b).start()
                return 0

            lax.fori_loop(0, N_K // 2, jbody, 0)

            if idx < N_DEV - 1:
                w_load(cols[idx + 1], 0, 0).start()
                w_load(cols[idx + 1], 1, 1).start()

            slot = idx % 2
            if idx >= 2:
                rdma_by_slot[slot].wait_send()
            for mi in range(2):
                rows = slice(mi * 1024, (mi + 1) * 1024)
                a = acc_ref[rows, :]
                send_bufs[slot, rows, :] = (
                    a * (1.0 / (1.0 + jnp.exp(-a)))
                ).astype(jnp.bfloat16)
            if idx < N_DEV - 1:
                rdma = pltpu.make_async_remote_copy(
                    src_ref=send_bufs.at[slot],
                    dst_ref=my_out_rows(),
                    send_sem=send_sems.at[slot],
                    recv_sem=recv_sems.at[idx],
                    device_id=(col,),
                    device_id_type=pl.DeviceIdType.MESH,
                )
                rdma.start()
                rdma_by_slot[slot] = rdma
            else:
                pltpu.make_async_copy(
                    send_bufs.at[slot], my_out_rows(), local_sem
                ).start()

        rdma_by_slot[0].wait_send()
        pltpu.make_async_copy(
            send_bufs.at[1], my_out_rows(), local_sem
        ).wait()
        for d in range(1, N_DEV):
            src = (my - d) % N_DEV
            pltpu.make_async_remote_copy(
                src_ref=send_bufs.at[0],
                dst_ref=out_ref.at[pl.ds(src * M_PER, M_PER), :],
                send_sem=send_sems.at[0],
                recv_sem=recv_sems.at[d - 1],
                device_id=(src,),
                device_id_type=pl.DeviceIdType.MESH,
            ).wait_recv()

    out_shape = jax.ShapeDtypeStruct((N_DEV * M_PER, N_PER), jnp.bfloat16)
    return pl.pallas_call(
        body,
        out_shape=out_shape,
        in_specs=[
            pl.BlockSpec(memory_space=pl.ANY),
            pl.BlockSpec(memory_space=pl.ANY),
        ],
        out_specs=pl.BlockSpec(memory_space=pl.ANY),
        scratch_shapes=[
            pltpu.VMEM((M_PER, K), jnp.bfloat16),
            pltpu.VMEM((2, M_PER, X_BLK), jnp.float32),
            pltpu.VMEM((2, K_BLK, N_PER), jnp.float32),
            pltpu.VMEM((M_PER, N_PER), jnp.float32),
            pltpu.VMEM((2, M_PER, N_PER), jnp.bfloat16),
            pltpu.SemaphoreType.DMA((2,)),
            pltpu.SemaphoreType.DMA((2,)),
            pltpu.SemaphoreType.DMA((2,)),
            pltpu.SemaphoreType.DMA((3,)),
            pltpu.SemaphoreType.DMA,
        ],
        compiler_params=pltpu.CompilerParams(
            collective_id=0,
            vmem_limit_bytes=64 * 1024 * 1024,
        ),
    )(x, w_mat)


# device time: 198077 ns/iter; 1.1236x vs baseline; 1.1236x over previous
import jax
import jax.numpy as jnp
from jax import lax
from jax.experimental import pallas as pl
from jax.experimental.pallas import tpu as pltpu

N_DEV = 4
M_PER = 2048
K = 8192
N_PER = 1024
K_BLK = 512
N_K = K // K_BLK


def kernel(x, w_mat):
    def body(x_hbm, w_hbm, out_ref, x_vmem, xf32_bufs, w_bufs, acc_ref,
             send_bufs, x_sems, w_sems, send_sems, recv_sems, local_sem):
        my = lax.axis_index("i")

        barrier = pltpu.get_barrier_semaphore()
        for d in range(1, N_DEV):
            pl.semaphore_signal(
                barrier, inc=1,
                device_id=((my + d) % N_DEV,),
                device_id_type=pl.DeviceIdType.MESH,
            )
        pl.semaphore_wait(barrier, N_DEV - 1)

        def x_load(k, b):
            return pltpu.make_async_copy(
                x_hbm.at[:, pl.ds(k * K_BLK, K_BLK)],
                xf32_bufs.at[b],
                x_sems.at[b],
            )

        def w_load(col, k, b):
            return pltpu.make_async_copy(
                w_hbm.at[pl.ds(k * K_BLK, K_BLK), pl.ds(col * N_PER, N_PER)],
                w_bufs.at[b],
                w_sems.at[b],
            )

        def my_out_rows():
            return out_ref.at[pl.ds(my * M_PER, M_PER), :]

        def k_step(idx, col, k, b, assign, guard):
            if idx == 0:
                x_load(k, b).wait()
                x_vmem[:, pl.ds(k * K_BLK, K_BLK)] = (
                    xf32_bufs[b].astype(jnp.bfloat16)
                )
                if guard:
                    @pl.when(k + 2 < N_K)
                    def _():
                        x_load(k + 2, b).start()
                else:
                    x_load(k + 2, b).start()

            w_load(col, k, b).wait()
            wt = w_bufs[b].astype(jnp.bfloat16)
            for mi in range(2):
                rows = slice(mi * 1024, (mi + 1) * 1024)
                part = lax.dot_general(
                    x_vmem[rows, pl.ds(k * K_BLK, K_BLK)],
                    wt,
                    (((1,), (0,)), ((), ())),
                    preferred_element_type=jnp.float32,
                )
                if assign:
                    acc_ref[rows, :] = part
                else:
                    acc_ref[rows, :] += part

            if guard:
                @pl.when(k + 2 < N_K)
                def _():
                    w_load(col, k + 2, b).start()
            else:
                w_load(col, k + 2, b).start()

        cols = [(my + d) % N_DEV for d in range(1, N_DEV)] + [my]
        rdma_by_slot = {}
        for idx in range(N_DEV):
            col = cols[idx]
            if idx == 0:
                w_load(col, 0, 0).start()
                w_load(col, 1, 1).start()
                x_load(0, 0).start()
                x_load(1, 1).start()

            k_step(idx, col, 0, 0, assign=True, guard=False)
            k_step(idx, col, 1, 1, assign=False, guard=False)

            def jbody(j, _, idx=idx, col=col):
                for b in (0, 1):
                    k_step(idx, col, 2 * j + b, b, assign=False, guard=True)
                return 0

            lax.fori_loop(1, N_K // 2, jbody, 0)

            if idx < N_DEV - 1:
                w_load(cols[idx + 1], 0, 0).start()
                w_load(cols[idx + 1], 1, 1).start()

            slot = idx % 2
            if idx >= 2:
                rdma_by_slot[slot].wait_send()
            for mi in range(2):
                rows = slice(mi * 1024, (mi + 1) * 1024)
                a = acc_ref[rows, :]
                send_bufs[slot, rows, :] = (
                    a * (1.0 / (1.0 + jnp.exp(-a)))
                ).astype(jnp.bfloat16)
            if idx < N_DEV - 1:
                rdma = pltpu.make_async_remote_copy(
                    src_ref=send_bufs.at[slot],
                    dst_ref=my_out_rows(),
                    send_sem=send_sems.at[slot],
                    recv_sem=recv_sems.at[idx],
                    device_id=(col,),
                    device_id_type=pl.DeviceIdType.MESH,
                )
                rdma.start()
                rdma_by_slot[slot] = rdma
            else:
                pltpu.make_async_copy(
                    send_bufs.at[slot], my_out_rows(), local_sem
                ).start()

        rdma_by_slot[0].wait_send()
        pltpu.make_async_copy(
            send_bufs.at[1], my_out_rows(), local_sem
        ).wait()
        for d in range(1, N_DEV):
            src = (my - d) % N_DEV
            pltpu.make_async_remote_copy(
                src_ref=send_bufs.at[0],
                dst_ref=out_ref.at[pl.ds(src * M_PER, M_PER), :],
                send_sem=send_sems.at[0],
                recv_sem=recv_sems.at[d - 1],
                device_id=(src,),
                device_id_type=pl.DeviceIdType.MESH,
            ).wait_recv()

    out_shape = jax.ShapeDtypeStruct((N_DEV * M_PER, N_PER), jnp.bfloat16)
    return pl.pallas_call(
        body,
        out_shape=out_shape,
        in_specs=[
            pl.BlockSpec(memory_space=pl.ANY),
            pl.BlockSpec(memory_space=pl.ANY),
        ],
        out_specs=pl.BlockSpec(memory_space=pl.ANY),
        scratch_shapes=[
            pltpu.VMEM((M_PER, K), jnp.bfloat16),
            pltpu.VMEM((2, M_PER, K_BLK), jnp.float32),
            pltpu.VMEM((2, K_BLK, N_PER), jnp.float32),
            pltpu.VMEM((M_PER, N_PER), jnp.float32),
            pltpu.VMEM((2, M_PER, N_PER), jnp.bfloat16),
            pltpu.SemaphoreType.DMA((2,)),
            pltpu.SemaphoreType.DMA((2,)),
            pltpu.SemaphoreType.DMA((2,)),
            pltpu.SemaphoreType.DMA((3,)),
            pltpu.SemaphoreType.DMA,
        ],
        compiler_params=pltpu.CompilerParams(
            collective_id=0,
            vmem_limit_bytes=64 * 1024 * 1024,
        ),
    )(x, w_mat)
